# baseline (speedup 1.0000x reference)
import jax
import jax.numpy as jnp
from jax.experimental import pallas as pl

_N = 2048


def _proj_body(pe_ref, wg_ref, bg_ref, out_ref):
    out_ref[...] = (
        jnp.dot(pe_ref[...], wg_ref[...], preferred_element_type=jnp.float32)
        + bg_ref[...]
    )


def kernel(edge_pe_index, edge_pe, num_nodes, W, b, gate):
    P, D = edge_pe.shape
    H = W.shape[1]
    g = jax.nn.sigmoid(gate)
    wg = (W * g[None, :]).astype(jnp.float32)
    bg = (b * g)[None, :].astype(jnp.float32)

    blk = 8192
    vals = pl.pallas_call(
        _proj_body,
        out_shape=jax.ShapeDtypeStruct((P, H), jnp.float32),
        grid=(P // blk,),
        in_specs=[
            pl.BlockSpec((blk, D), lambda i: (i, 0)),
            pl.BlockSpec((D, H), lambda i: (0, 0)),
            pl.BlockSpec((1, H), lambda i: (0, 0)),
        ],
        out_specs=pl.BlockSpec((blk, H), lambda i: (i, 0)),
    )(edge_pe, wg, bg)

    row = (edge_pe_index[0].astype(jnp.int32)) % num_nodes.astype(jnp.int32)
    col = (edge_pe_index[1].astype(jnp.int32)) % num_nodes.astype(jnp.int32)
    key = row * _N + col
    order = jnp.argsort(key, stable=True)
    ks = key[order]
    is_last = jnp.concatenate([ks[1:] != ks[:-1], jnp.array([True])])
    rs = ks // _N
    cs = ks % _N
    rs = jnp.where(is_last, rs, _N)
    vals_s = vals[order]
    bias = jnp.zeros((H, _N, _N), jnp.float32)
    bias = bias.at[:, rs, cs].set(vals_s.T, mode="drop")
    return bias
